# baseline (speedup 1.0000x reference)
import functools

import jax
import jax.numpy as jnp
from jax import lax
from jax.experimental import pallas as pl
from jax.experimental.pallas import tpu as pltpu
from jax.experimental.pallas import tpu_sc as plsc

D_MODEL = 128
SEQ = 200
BATCH = 4096
NUM_ROWS = BATCH * SEQ
NUM_CORES = 2
NUM_SUBCORES = 16
NUM_WORKERS = NUM_CORES * NUM_SUBCORES
ROWS_PER_WORKER = NUM_ROWS // NUM_WORKERS
CHUNK = 64
NUM_CHUNKS = ROWS_PER_WORKER // CHUNK
LANES = 16
NBUF = 8
_MAX_T0 = SEQ - (SEQ % CHUNK or CHUNK)
POS_ROWS = _MAX_T0 + CHUNK

assert NUM_CHUNKS % NBUF == 0 and NBUF >= 2 and CHUNK % 8 == 0 and CHUNK <= 128


@jax.jit
def _emb_lookup(x_flat, token_table, pos_table):
    mesh = plsc.VectorSubcoreMesh(
        core_axis_name="c", subcore_axis_name="s",
        num_cores=NUM_CORES, num_subcores=NUM_SUBCORES,
    )

    @functools.partial(
        pl.kernel,
        mesh=mesh,
        out_type=jax.ShapeDtypeStruct((NUM_ROWS, D_MODEL), jnp.float32),
        scratch_types=[
            pltpu.VMEM((ROWS_PER_WORKER,), jnp.int32),
            pltpu.VMEM((POS_ROWS, D_MODEL), jnp.float32),
            [pltpu.VMEM((CHUNK, D_MODEL), jnp.float32)] * NBUF,
            [pltpu.SemaphoreType.DMA] * NBUF,
            [pltpu.SemaphoreType.DMA] * NBUF,
        ],
    )
    def k(x_hbm, tok_hbm, pos_hbm, out_hbm, idx_v, pos_v, rows, gsem, osem):
        wid = lax.axis_index("s") * NUM_CORES + lax.axis_index("c")
        base = pl.multiple_of(wid * ROWS_PER_WORKER, CHUNK)

        pltpu.sync_copy(x_hbm.at[pl.ds(base, ROWS_PER_WORKER)], idx_v)
        pltpu.sync_copy(pos_hbm.at[pl.ds(0, SEQ)], pos_v.at[pl.ds(0, SEQ)])
        pltpu.sync_copy(pos_hbm.at[pl.ds(0, POS_ROWS - SEQ)],
                        pos_v.at[pl.ds(SEQ, POS_ROWS - SEQ)])

        def gather_start(k_, buf):
            start = pl.multiple_of(k_ * CHUNK, CHUNK)
            pltpu.async_copy(
                tok_hbm.at[idx_v.at[pl.ds(start, CHUNK)]], rows[buf], gsem[buf]
            )

        def gather_wait(k_, buf):
            start = pl.multiple_of(k_ * CHUNK, CHUNK)
            pltpu.make_async_copy(
                tok_hbm.at[idx_v.at[pl.ds(start, CHUNK)]], rows[buf], gsem[buf]
            ).wait()

        def out_start(k_, buf):
            start = pl.multiple_of(k_ * CHUNK, CHUNK)
            pltpu.async_copy(
                rows[buf], out_hbm.at[pl.ds(base + start, CHUNK)], osem[buf]
            )

        def out_wait(k_, buf):
            start = pl.multiple_of(k_ * CHUNK, CHUNK)
            pltpu.make_async_copy(
                rows[buf], out_hbm.at[pl.ds(base + start, CHUNK)], osem[buf]
            ).wait()

        def add_pos(k_, buf):
            t0 = lax.rem(k_ * CHUNK, SEQ)
            rbuf = rows[buf]

            @plsc.parallel_loop(0, CHUNK, unroll=4)
            def _(i):
                t = t0 + i
                for j in range(D_MODEL // LANES):
                    pv = pos_v[t, pl.ds(j * LANES, LANES)]
                    plsc.addupdate(rbuf.at[i, pl.ds(j * LANES, LANES)], pv)

        def consume(kc, b):
            gather_wait(kc, b)
            add_pos(kc, b)
            out_start(kc, b)

        gather_start(0, 0)
        for kc in range(NBUF - 1):
            gather_start(kc + 1, (kc + 1) % NBUF)
            consume(kc, kc % NBUF)

        def loop_body(it, carry):
            c = NBUF - 1 + it * NBUF
            for db in range(NBUF):
                kc = c + db
                b = (NBUF - 1 + db) % NBUF
                bn = db % NBUF
                out_wait(kc + 1 - NBUF, bn)
                gather_start(kc + 1, bn)
                consume(kc, b)
            return carry

        lax.fori_loop(0, (NUM_CHUNKS - NBUF) // NBUF, loop_body, 0)

        consume(NUM_CHUNKS - 1, (NUM_CHUNKS - 1) % NBUF)
        for kc in range(NUM_CHUNKS - NBUF, NUM_CHUNKS):
            out_wait(kc, kc % NBUF)

    return k(x_flat, token_table, pos_table)


def kernel(x, token_table, pos_table):
    x_flat = x.reshape(-1).astype(jnp.int32)
    out = _emb_lookup(x_flat, token_table, pos_table)
    return out.reshape(BATCH, SEQ, D_MODEL)


# speedup vs baseline: 1.2098x; 1.2098x over previous
import functools

import jax
import jax.numpy as jnp
from jax import lax
from jax.experimental import pallas as pl
from jax.experimental.pallas import tpu as pltpu
from jax.experimental.pallas import tpu_sc as plsc

D_MODEL = 128
SEQ = 200
BATCH = 4096
NUM_ROWS = BATCH * SEQ
NUM_CORES = 2
NUM_SUBCORES = 16
NUM_WORKERS = NUM_CORES * NUM_SUBCORES
ROWS_PER_WORKER = NUM_ROWS // NUM_WORKERS
CHUNK = 128
NUM_CHUNKS = ROWS_PER_WORKER // CHUNK
LANES = 16
NBUF = 4
POS_ROWS = SEQ

assert NUM_CHUNKS % NBUF == 0 and NBUF >= 2 and CHUNK % 8 == 0 and CHUNK <= 128


@jax.jit
def _emb_lookup(x_flat, token_table, pos_table):
    mesh = plsc.VectorSubcoreMesh(
        core_axis_name="c", subcore_axis_name="s",
        num_cores=NUM_CORES, num_subcores=NUM_SUBCORES,
    )

    @functools.partial(
        pl.kernel,
        mesh=mesh,
        out_type=jax.ShapeDtypeStruct((NUM_ROWS, D_MODEL), jnp.float32),
        scratch_types=[
            pltpu.VMEM((ROWS_PER_WORKER,), jnp.int32),
            pltpu.VMEM((POS_ROWS, D_MODEL), jnp.float32),
            [pltpu.VMEM((CHUNK, D_MODEL), jnp.float32)] * NBUF,
            [pltpu.SemaphoreType.DMA] * NBUF,
            [pltpu.SemaphoreType.DMA] * NBUF,
        ],
    )
    def k(x_hbm, tok_hbm, pos_hbm, out_hbm, idx_v, pos_v, rows, gsem, osem):
        wid = lax.axis_index("s") * NUM_CORES + lax.axis_index("c")
        base = pl.multiple_of(wid * ROWS_PER_WORKER, CHUNK)

        pltpu.sync_copy(x_hbm.at[pl.ds(base, ROWS_PER_WORKER)], idx_v)
        pltpu.sync_copy(pos_hbm.at[pl.ds(0, SEQ)], pos_v)

        def gather_start(k_, buf):
            start = pl.multiple_of(k_ * CHUNK, CHUNK)
            pltpu.async_copy(
                tok_hbm.at[idx_v.at[pl.ds(start, CHUNK)]], rows[buf], gsem[buf]
            )

        def gather_wait(k_, buf):
            start = pl.multiple_of(k_ * CHUNK, CHUNK)
            pltpu.make_async_copy(
                tok_hbm.at[idx_v.at[pl.ds(start, CHUNK)]], rows[buf], gsem[buf]
            ).wait()

        def out_start(k_, buf):
            start = pl.multiple_of(k_ * CHUNK, CHUNK)
            pltpu.async_copy(
                rows[buf], out_hbm.at[pl.ds(base + start, CHUNK)], osem[buf]
            )

        def out_wait(k_, buf):
            start = pl.multiple_of(k_ * CHUNK, CHUNK)
            pltpu.make_async_copy(
                rows[buf], out_hbm.at[pl.ds(base + start, CHUNK)], osem[buf]
            ).wait()

        def add_pos(k_, buf):
            t0 = lax.rem(k_ * CHUNK, SEQ)
            rbuf = rows[buf]

            @plsc.parallel_loop(0, CHUNK, unroll=4)
            def _(i):
                t = lax.rem(t0 + i, SEQ)
                for j in range(D_MODEL // LANES):
                    pv = pos_v[t, pl.ds(j * LANES, LANES)]
                    plsc.addupdate(rbuf.at[i, pl.ds(j * LANES, LANES)], pv)

        def consume(kc, b):
            gather_wait(kc, b)
            add_pos(kc, b)
            out_start(kc, b)

        gather_start(0, 0)
        for kc in range(NBUF - 1):
            gather_start(kc + 1, (kc + 1) % NBUF)
            consume(kc, kc % NBUF)

        def loop_body(it, carry):
            c = NBUF - 1 + it * NBUF
            for db in range(NBUF):
                kc = c + db
                b = (NBUF - 1 + db) % NBUF
                bn = db % NBUF
                out_wait(kc + 1 - NBUF, bn)
                gather_start(kc + 1, bn)
                consume(kc, b)
            return carry

        lax.fori_loop(0, (NUM_CHUNKS - NBUF) // NBUF, loop_body, 0)

        consume(NUM_CHUNKS - 1, (NUM_CHUNKS - 1) % NBUF)
        for kc in range(NUM_CHUNKS - NBUF, NUM_CHUNKS):
            out_wait(kc, kc % NBUF)

    return k(x_flat, token_table, pos_table)


def kernel(x, token_table, pos_table):
    x_flat = x.reshape(-1).astype(jnp.int32)
    out = _emb_lookup(x_flat, token_table, pos_table)
    return out.reshape(BATCH, SEQ, D_MODEL)
